# baseline (speedup 1.0000x reference)
import jax
import jax.numpy as jnp
from jax.experimental import pallas as pl
from jax.experimental.pallas import tpu as pltpu

_N, _C, _H, _W = 32, 128, 56, 56
_S = _H * _W
_CNT = _N * _S
_EPS = 1e-5
_B = 2
_G = _N // _B


def _stats_kernel(x0_ref, x1_ref, x2_ref, x3_ref, out_ref):
    for j, ref in enumerate((x0_ref, x1_ref, x2_ref, x3_ref)):
        x = ref[...]
        out_ref[0, :, j:j + 1] = jnp.sum(x, axis=(0, 2))[:, None]
        out_ref[0, :, j + 4:j + 5] = jnp.sum(x * x, axis=(0, 2))[:, None]


def _main_kernel(x0_ref, x1_ref, x2_ref, x3_ref, p_ref, w_ref, b_ref,
                 wmat_ref, out_ref):
    tot = jnp.sum(p_ref[...], axis=0)
    mean = tot[:, 0:4] * (1.0 / _CNT)
    ex2 = tot[:, 4:8] * (1.0 / _CNT)
    var = ex2 - mean * mean
    inv = jax.lax.rsqrt(var + _EPS)
    scale = w_ref[...] * inv
    shift = b_ref[...] - mean * scale
    for i in range(_B):
        acc = jnp.zeros((_C, _S), dtype=jnp.float32)
        for j, ref in enumerate((x0_ref, x1_ref, x2_ref, x3_ref)):
            x = ref[i]
            y = jnp.maximum(x * scale[:, j:j + 1] + shift[:, j:j + 1],
                            0.0).astype(jnp.bfloat16)
            acc = acc + jnp.dot(wmat_ref[:, j * _C:(j + 1) * _C], y,
                                preferred_element_type=jnp.float32)
        out_ref[i] = acc


@jax.jit
def kernel(x0, x1, x2, x3, norm_weight, norm_bias, conv_weight):
    xs = [x.reshape(_N, _C, _S) for x in (x0, x1, x2, x3)]
    wg = norm_weight.reshape(4, _C).T
    bg = norm_bias.reshape(4, _C).T
    wmat = conv_weight.reshape(_C, 4 * _C).astype(jnp.bfloat16)

    x_spec = pl.BlockSpec((_B, _C, _S), lambda i: (i, 0, 0))

    partials = pl.pallas_call(
        _stats_kernel,
        grid=(_G,),
        in_specs=[x_spec] * 4,
        out_specs=pl.BlockSpec((1, _C, 8), lambda i: (i, 0, 0)),
        out_shape=jax.ShapeDtypeStruct((_G, _C, 8), jnp.float32),
        compiler_params=pltpu.CompilerParams(
            dimension_semantics=("arbitrary",),
            vmem_limit_bytes=50 * 1024 * 1024),
    )(*xs)

    out = pl.pallas_call(
        _main_kernel,
        grid=(_G,),
        in_specs=[x_spec] * 4 + [
            pl.BlockSpec((_G, _C, 8), lambda i: (0, 0, 0)),
            pl.BlockSpec((_C, 4), lambda i: (0, 0)),
            pl.BlockSpec((_C, 4), lambda i: (0, 0)),
            pl.BlockSpec((_C, 4 * _C), lambda i: (0, 0)),
        ],
        out_specs=pl.BlockSpec((_B, _C, _S), lambda i: (i, 0, 0)),
        out_shape=jax.ShapeDtypeStruct((_N, _C, _S), jnp.float32),
        compiler_params=pltpu.CompilerParams(
            dimension_semantics=("arbitrary",),
            vmem_limit_bytes=50 * 1024 * 1024),
    )(*xs, partials, wg, bg, wmat)

    return out.reshape(_N, _C, _H, _W)


# speedup vs baseline: 1.0131x; 1.0131x over previous
import jax
import jax.numpy as jnp
from jax.experimental import pallas as pl
from jax.experimental.pallas import tpu as pltpu

_N, _C, _H, _W = 32, 128, 56, 56
_S = _H * _W
_CNT = _N * _S
_EPS = 1e-5
_B = 2
_G = _N // _B


def _fused_kernel(x0_ref, x1_ref, x2_ref, x3_ref, w_ref, b_ref, wmat_ref,
                  out_ref, acc_ref):
    p = pl.program_id(0)
    i = pl.program_id(1)

    @pl.when(jnp.logical_and(p == 0, i == 0))
    def _():
        acc_ref[...] = jnp.zeros_like(acc_ref)

    @pl.when(p == 0)
    def _():
        for j, ref in enumerate((x0_ref, x1_ref, x2_ref, x3_ref)):
            x = ref[...]
            acc_ref[:, j:j + 1] += jnp.sum(x, axis=(0, 2))[:, None]
            acc_ref[:, j + 4:j + 5] += jnp.sum(x * x, axis=(0, 2))[:, None]

    @pl.when(p == 1)
    def _():
        tot = acc_ref[...]
        mean = tot[:, 0:4] * (1.0 / _CNT)
        ex2 = tot[:, 4:8] * (1.0 / _CNT)
        var = ex2 - mean * mean
        inv = jax.lax.rsqrt(var + _EPS)
        scale = w_ref[...] * inv
        shift = b_ref[...] - mean * scale
        for b in range(_B):
            acc = jnp.zeros((_C, _S), dtype=jnp.float32)
            for j, ref in enumerate((x0_ref, x1_ref, x2_ref, x3_ref)):
                x = ref[b]
                y = jnp.maximum(x * scale[:, j:j + 1] + shift[:, j:j + 1], 0.0)
                acc = acc + jnp.dot(wmat_ref[:, j * _C:(j + 1) * _C], y,
                                    preferred_element_type=jnp.float32)
            out_ref[b] = acc


@jax.jit
def kernel(x0, x1, x2, x3, norm_weight, norm_bias, conv_weight):
    xs = [x.reshape(_N, _C, _S) for x in (x0, x1, x2, x3)]
    wg = norm_weight.reshape(4, _C).T
    bg = norm_bias.reshape(4, _C).T
    wmat = conv_weight.reshape(_C, 4 * _C)

    x_spec = pl.BlockSpec((_B, _C, _S), lambda p, i: (i, 0, 0))

    out = pl.pallas_call(
        _fused_kernel,
        grid=(2, _G),
        in_specs=[x_spec] * 4 + [
            pl.BlockSpec((_C, 4), lambda p, i: (0, 0)),
            pl.BlockSpec((_C, 4), lambda p, i: (0, 0)),
            pl.BlockSpec((_C, 4 * _C), lambda p, i: (0, 0)),
        ],
        out_specs=pl.BlockSpec((_B, _C, _S), lambda p, i: (p * i, 0, 0)),
        out_shape=jax.ShapeDtypeStruct((_N, _C, _S), jnp.float32),
        scratch_shapes=[pltpu.VMEM((_C, 8), jnp.float32)],
        compiler_params=pltpu.CompilerParams(
            dimension_semantics=("arbitrary", "arbitrary"),
            vmem_limit_bytes=50 * 1024 * 1024),
    )(*xs, wg, bg, wmat)

    return out.reshape(_N, _C, _H, _W)
